# baseline (speedup 1.0000x reference)
import jax
import jax.numpy as jnp
from jax.experimental import pallas as pl
from jax.experimental.pallas import tpu as pltpu

FEA = 768
DK = 64
H = 12
L = 2048
SCALE = 1.0 / 8.0
BL = 512
NB = L // BL
BF = jnp.bfloat16


def _head_onehot():
    ci = jax.lax.broadcasted_iota(jnp.int32, (FEA, H), 0)
    hi = jax.lax.broadcasted_iota(jnp.int32, (FEA, H), 1)
    return (ci // DK == hi).astype(jnp.float32)


def _head_onehot_t():
    hi = jax.lax.broadcasted_iota(jnp.int32, (H, FEA), 0)
    ci = jax.lax.broadcasted_iota(jnp.int32, (H, FEA), 1)
    return (ci // DK == hi).astype(jnp.float32)


def _mm_t(x, w):
    return jax.lax.dot_general(x, w, (((1,), (1,)), ((), ())),
                               preferred_element_type=jnp.float32)


def _mm(x, w):
    return jax.lax.dot_general(x, w, (((1,), (0,)), ((), ())),
                               preferred_element_type=jnp.float32)


def _body(qx, kx, vx, qe, wq, wk, wv, wo, bq, bk, bv, bo, out,
          Ks, Vs, sall_s, qg_s, u_s, d_s):
    p = pl.program_id(0)
    j = pl.program_id(1)
    base = j * BL
    ET = _head_onehot_t()

    @pl.when(p == 0)
    def _proj():
        wqb = wq[...].astype(BF)
        wkb = wk[...].astype(BF)
        wvb = wv[...].astype(BF)
        kv = _mm_t(kx[...].astype(BF), wkb) + bk[...]
        vv = _mm_t(vx[...].astype(BF), wvb) + bv[...]
        Ks[pl.ds(base, BL), :] = kv.astype(BF)
        Vs[pl.ds(base, BL), :] = vv.astype(BF)
        part = jnp.sum(vv, axis=0, keepdims=True)

        @pl.when(j == 0)
        def _():
            qg_s[...] = (_mm_t(qe[...].astype(BF), wqb) + bq[...]) * SCALE
            sall_s[...] = part

        @pl.when(j > 0)
        def _():
            sall_s[...] += part

        qg = qg_s[...]
        G = jnp.concatenate([ET * qg[0:1, :], ET * qg[1:2, :]],
                            axis=0).astype(BF)
        s = _mm_t(G, kv.astype(BF))
        es = jnp.exp(s)
        c = _mm(es.astype(BF), vv.astype(BF))
        dpart = jnp.sum(es, axis=1, keepdims=True)

        @pl.when(j == 0)
        def _():
            u_s[...] = c
            d_s[...] = jnp.broadcast_to(dpart, (2 * H, 128))

        @pl.when(j > 0)
        def _():
            u_s[...] += c
            d_s[...] += jnp.broadcast_to(dpart, (2 * H, 128))

    @pl.when(p == 1)
    def _attn():
        E = _head_onehot()

        Qb = (_mm_t(qx[...].astype(BF), wq[...].astype(BF)) + bq[...]) * SCALE
        wob = wo[...].astype(BF)

        k0 = Ks[0:16, :][0:1].astype(jnp.float32)
        kL = Ks[L - 16:L, :][15:16].astype(jnp.float32)
        v0 = Vs[0:16, :][0:1].astype(jnp.float32)
        vL = Vs[L - 16:L, :][15:16].astype(jnp.float32)
        sall = sall_s[...]

        kblk = Ks[pl.ds(base, BL), :].astype(jnp.float32)
        vblk = Vs[pl.ds(base, BL), :].astype(jnp.float32)
        prev_w = jnp.maximum(j * (BL // 16) - 1, 0) * 16
        next_w = jnp.minimum((j + 1) * (BL // 16), L // 16 - 1) * 16
        kprev = Ks[pl.ds(prev_w, 16), :][15:16].astype(jnp.float32)
        knext = Ks[pl.ds(next_w, 16), :][0:1].astype(jnp.float32)
        vprev = Vs[pl.ds(prev_w, 16), :][15:16].astype(jnp.float32)
        vnext = Vs[pl.ds(next_w, 16), :][0:1].astype(jnp.float32)
        km1 = jnp.concatenate([kprev, kblk[:BL - 1, :]], axis=0)
        kp1 = jnp.concatenate([kblk[1:, :], knext], axis=0)
        vm1 = jnp.concatenate([vprev, vblk[:BL - 1, :]], axis=0)
        vp1 = jnp.concatenate([vblk[1:, :], vnext], axis=0)

        G2 = jnp.concatenate([ET * k0, ET * kL], axis=0)
        e0L = _mm_t(Qb, G2)
        x0 = jnp.exp(e0L[:, :H])
        xL = jnp.exp(e0L[:, H:])
        xd = jnp.exp(_mm(Qb * kblk, E))
        xsub = jnp.exp(_mm(Qb * km1, E))
        xsup = jnp.exp(_mm(Qb * kp1, E))

        gi = base + jax.lax.broadcasted_iota(jnp.int32, (BL, 1), 0)
        msub = (gi != 1).astype(jnp.float32)
        msup = (gi != L - 2).astype(jnp.float32)

        denom = (x0 + xL + xd + msub * xsub + msup * xsup
                 + (jnp.float32(L - 3) - msub - msup))
        recip = 1.0 / denom

        X3 = jnp.concatenate(
            [recip, recip * (x0 - 1.0), recip * (xL - 1.0)], axis=1)
        W3 = jnp.concatenate([ET * sall, ET * v0, ET * vL], axis=0)
        z = (_mm(X3, W3)
             + _mm(recip * (xd - 1.0), ET) * vblk
             + _mm(recip * msub * (xsub - 1.0), ET) * vm1
             + _mm(recip * msup * (xsup - 1.0), ET) * vp1)

        out[...] = _mm_t(z.astype(BF), wob) + bo[...]

        @pl.when(j == 0)
        def _():
            w0 = u_s[...][0:H] / d_s[...][0:H, 0:1]
            z0 = jnp.sum(ET * w0, axis=0, keepdims=True)
            out[0:1, :] = _mm_t(z0.astype(BF), wob) + bo[...]

        @pl.when(j == NB - 1)
        def _():
            wL = u_s[...][H:] / d_s[...][H:, 0:1]
            zL = jnp.sum(ET * wL, axis=0, keepdims=True)
            out[BL - 1:BL, :] = _mm_t(zL.astype(BF), wob) + bo[...]


def kernel(qx, kx, vx, WQ_w, WQ_b, WK_w, WK_b, WV_w, WV_b, WO_w, WO_b):
    q2 = qx.reshape(L, FEA)
    k2 = kx.reshape(L, FEA)
    v2 = vx.reshape(L, FEA)
    qe = jnp.concatenate([q2[0:1, :], q2[L - 1:L, :]], axis=0)
    bq = WQ_b.reshape(1, FEA)
    bk = WK_b.reshape(1, FEA)
    bv = WV_b.reshape(1, FEA)
    bo = WO_b.reshape(1, FEA)

    q_blk = pl.BlockSpec((BL, FEA), lambda p, j: (j, 0))
    kv_blk = pl.BlockSpec((BL, FEA), lambda p, j: (j * (1 - p), 0))
    full_w = pl.BlockSpec((FEA, FEA), lambda p, j: (0, 0))
    full_b = pl.BlockSpec((1, FEA), lambda p, j: (0, 0))
    qe_blk = pl.BlockSpec((2, FEA), lambda p, j: (0, 0))
    out_blk = pl.BlockSpec((BL, FEA), lambda p, j: (j * p, 0))

    out = pl.pallas_call(
        _body,
        grid=(2, NB),
        in_specs=[q_blk, kv_blk, kv_blk, qe_blk, full_w, full_w, full_w,
                  full_w, full_b, full_b, full_b, full_b],
        out_specs=out_blk,
        out_shape=jax.ShapeDtypeStruct((L, FEA), jnp.float32),
        compiler_params=pltpu.CompilerParams(vmem_limit_bytes=100 * 1024 * 1024),
        scratch_shapes=[
            pltpu.VMEM((L, FEA), BF),
            pltpu.VMEM((L, FEA), BF),
            pltpu.VMEM((1, FEA), jnp.float32),
            pltpu.VMEM((2, FEA), jnp.float32),
            pltpu.VMEM((2 * H, FEA), jnp.float32),
            pltpu.VMEM((2 * H, 128), jnp.float32),
        ],
    )(q2, k2, v2, qe, WQ_w, WK_w, WV_w, WO_w, bq, bk, bv, bo)

    return out.reshape(1, L, FEA)
